# Initial kernel scaffold; baseline (speedup 1.0000x reference)
import jax
import jax.numpy as jnp
from jax.experimental import pallas as pl


def kernel(x, W_enc, b_enc, W_dec):
    raise NotImplementedError("write your pallas kernel here")


# speedup vs baseline: 14.1442x; 14.1442x over previous
import functools

import jax
import jax.numpy as jnp
from jax.experimental import pallas as pl
from jax.experimental.pallas import tpu as pltpu

K = 64


def _encode_body(x_ref, w_ref, b_ref, pre_ref):
    acc = jax.lax.dot_general(
        x_ref[...], w_ref[...], (((1,), (1,)), ((), ())),
        preferred_element_type=jnp.float32)
    pre_ref[...] = acc + b_ref[...]


def _topk_body(pre_ref, lat_ref, key_ref):
    pre = pre_ref[...]
    ki = jax.lax.bitcast_convert_type(pre, jnp.int32)
    key = jnp.where(ki < 0, ki ^ 0x7FFFFFFF, ki)
    key_ref[...] = key
    nonneg = jnp.sum((key >= 0).astype(jnp.int32), axis=1, keepdims=True)
    int_min = jnp.int32(-2147483647 - 1)
    base = jnp.where(nonneg >= K, jnp.int32(0), int_min)

    def body(t, base):
        cand = base + (jnp.int32(1) << (30 - t))
        cnt = jnp.sum((key_ref[...] >= cand).astype(jnp.int32), axis=1,
                      keepdims=True)
        return jnp.where(cnt >= K, cand, base)

    base = jax.lax.fori_loop(0, 31, body, base)
    lat_ref[...] = jnp.where(key_ref[...] >= base, pre_ref[...], 0.0)


def _decode_body(lat_ref, wd_ref, out_ref):
    j = pl.program_id(0)
    i = pl.program_id(1)
    rb = lat_ref.shape[0]
    acc = jax.lax.dot_general(
        lat_ref[...].astype(jnp.bfloat16), wd_ref[...],
        (((1,), (1,)), ((), ())), preferred_element_type=jnp.float32)
    rows = pl.ds(i * rb, rb)

    @pl.when(j == 0)
    def _init():
        out_ref[rows, :] = acc

    @pl.when(j != 0)
    def _acc():
        out_ref[rows, :] = out_ref[rows, :] + acc


def kernel(x, W_enc, b_enc, W_dec):
    n, d_model = x.shape
    d_sae = W_enc.shape[0]

    rb_e = min(1024, n)
    cb_e = min(2048, d_sae)
    b2 = b_enc.reshape(1, d_sae)
    pre = pl.pallas_call(
        _encode_body,
        grid=(n // rb_e, d_sae // cb_e),
        in_specs=[
            pl.BlockSpec((rb_e, d_model), lambda i, j: (i, 0)),
            pl.BlockSpec((cb_e, d_model), lambda i, j: (j, 0)),
            pl.BlockSpec((1, cb_e), lambda i, j: (0, j)),
        ],
        out_specs=pl.BlockSpec((rb_e, cb_e), lambda i, j: (i, j)),
        out_shape=jax.ShapeDtypeStruct((n, d_sae), jnp.float32),
    )(x, W_enc, b2)

    rb_t = min(128, n)
    lat = pl.pallas_call(
        _topk_body,
        grid=(n // rb_t,),
        in_specs=[pl.BlockSpec((rb_t, d_sae), lambda i: (i, 0))],
        out_specs=pl.BlockSpec((rb_t, d_sae), lambda i: (i, 0)),
        out_shape=jax.ShapeDtypeStruct((n, d_sae), jnp.float32),
        scratch_shapes=[pltpu.VMEM((rb_t, d_sae), jnp.int32)],
    )(pre)

    rb_d = min(1024, n)
    cb_d = min(2048, d_sae)
    wd_bf = W_dec.astype(jnp.bfloat16)
    out = pl.pallas_call(
        _decode_body,
        grid=(d_sae // cb_d, n // rb_d),
        in_specs=[
            pl.BlockSpec((rb_d, cb_d), lambda j, i: (i, j)),
            pl.BlockSpec((d_model, cb_d), lambda j, i: (0, j)),
        ],
        out_specs=pl.BlockSpec((n, d_model), lambda j, i: (0, 0)),
        out_shape=jax.ShapeDtypeStruct((n, d_model), jnp.float32),
    )(lat, wd_bf)

    return (out, lat)
